# baseline (speedup 1.0000x reference)
import jax
import jax.numpy as jnp
from jax.experimental import pallas as pl
from jax.experimental.pallas import tpu as pltpu

_K = 10
_Q = 1024
_D = 128
_N = 100000
_NEG = -3.0e38
_IHUGE = 2**31 - 1

_CB = 2048
_NCHUNK = _CB // 128
_NCLS = 4
_NRANK = 3


def _fast_body(qn_ref, cn_ref, vals_out, idx_out, flag_out, tv, ti):
    j = pl.program_id(0)
    nb = pl.num_programs(0)

    @pl.when(j == 0)
    def _init():
        tv[...] = jnp.full(tv.shape, _NEG, jnp.float32)
        ti[...] = jnp.zeros(ti.shape, jnp.int32)

    qn = qn_ref[...]
    lane128 = jax.lax.broadcasted_iota(jnp.int32, (_Q, 128), 1)

    s_all = jax.lax.dot_general(qn, cn_ref[...], (((1,), (1,)), ((), ())),
                                preferred_element_type=jnp.float32)

    for p in range(_NCLS):
        t1, t2, t3 = tv[p, 0], tv[p, 1], tv[p, 2]
        i1, i2 = ti[p, 0], ti[p, 1]
        for t in range(p, _NCHUNK, _NCLS):
            g = j * _NCHUNK + t
            v = s_all[:, t * 128:(t + 1) * 128]
            v = jnp.where(g * 128 + lane128 < _N, v, _NEG)
            gi = jnp.full((_Q, 128), g, jnp.int32)
            b1 = v > t1
            b2 = v > t2
            b3 = v > t3
            t3 = jnp.where(b2, t2, jnp.where(b3, v, t3))
            t2 = jnp.where(b1, t1, jnp.where(b2, v, t2))
            i2 = jnp.where(b1, i1, jnp.where(b2, gi, i2))
            t1 = jnp.where(b1, v, t1)
            i1 = jnp.where(b1, gi, i1)
        tv[p, 0], tv[p, 1], tv[p, 2] = t1, t2, t3
        ti[p, 0], ti[p, 1] = i1, i2

    @pl.when(j == nb - 1)
    def _extract():
        lane = jax.lax.broadcasted_iota(jnp.int32, (_Q, 128), 1)
        cv = jnp.full((_Q, 128), _NEG, jnp.float32)
        ci = jnp.zeros((_Q, 128), jnp.int32)
        r = 0
        for p in range(_NCLS):
            for rank in range(2):
                av = tv[p, rank]
                ag = ti[p, rank] * 128 + lane
                for i in range(_K):
                    m = jnp.max(av, axis=1, keepdims=True)
                    eq = av == m
                    gidx = jnp.min(jnp.where(eq, ag, _IHUGE), axis=1,
                                   keepdims=True)
                    s = r * _K + i
                    cv = jnp.where(lane == s, m, cv)
                    ci = jnp.where(lane == s, gidx, ci)
                    av = jnp.where(eq & (ag == gidx), _NEG, av)
                r += 1
        ov = jnp.zeros((_Q, 128), jnp.float32)
        oi = jnp.zeros((_Q, 128), jnp.int32)
        m10 = None
        for i in range(_K):
            m = jnp.max(cv, axis=1, keepdims=True)
            eq = cv == m
            gidx = jnp.min(jnp.where(eq, ci, _IHUGE), axis=1, keepdims=True)
            ov = jnp.where(lane == i, m, ov)
            oi = jnp.where(lane == i, gidx, oi)
            cv = jnp.where(eq & (ci == gidx), _NEG, cv)
            m10 = m
        vals_out[...] = ov[:, :_K]
        idx_out[...] = oi[:, :_K]
        fl = jnp.zeros((_Q, 128), jnp.int32)
        for p in range(_NCLS):
            fl = fl | (tv[p, 2] >= m10).astype(jnp.int32)
        flag_out[...] = fl


def _naive_body(qn_ref, cn_ref, vals_out, idx_out, carry_v, carry_i):
    j = pl.program_id(0)
    nb = pl.num_programs(0)

    @pl.when(j == 0)
    def _init():
        carry_v[...] = jnp.full((_Q, 128), _NEG, jnp.float32)
        carry_i[...] = jnp.zeros((_Q, 128), jnp.int32)

    s = jax.lax.dot_general(qn_ref[...], cn_ref[...], (((1,), (1,)), ((), ())),
                            preferred_element_type=jnp.float32)
    col = j * _CB + jax.lax.broadcasted_iota(jnp.int32, (_Q, _CB), 1)
    s = jnp.where(col < _N, s, _NEG)

    lane128 = jax.lax.broadcasted_iota(jnp.int32, (_Q, 128), 1)
    bv = jnp.full((_Q, 128), _NEG, jnp.float32)
    bi = jnp.zeros((_Q, 128), jnp.int32)
    for i in range(_K):
        m = jnp.max(s, axis=1, keepdims=True)
        eq = s == m
        gidx = jnp.min(jnp.where(eq, col, _IHUGE), axis=1, keepdims=True)
        bv = jnp.where(lane128 == i, m, bv)
        bi = jnp.where(lane128 == i, gidx, bi)
        s = jnp.where(eq & (col == gidx), _NEG, s)

    mv = jnp.concatenate([carry_v[...], bv], axis=1)
    mi = jnp.concatenate([carry_i[...], bi], axis=1)
    nv = jnp.full((_Q, 128), _NEG, jnp.float32)
    ni = jnp.zeros((_Q, 128), jnp.int32)
    for i in range(_K):
        m = jnp.max(mv, axis=1, keepdims=True)
        eq = mv == m
        gidx = jnp.min(jnp.where(eq, mi, _IHUGE), axis=1, keepdims=True)
        nv = jnp.where(lane128 == i, m, nv)
        ni = jnp.where(lane128 == i, gidx, ni)
        mv = jnp.where(eq & (mi == gidx), _NEG, mv)
    carry_v[...] = nv
    carry_i[...] = ni

    @pl.when(j == nb - 1)
    def _finish():
        vals_out[...] = carry_v[...]
        idx_out[...] = carry_i[...]


def _run_fast(qn, cn):
    nb = pl.cdiv(_N, _CB)
    return pl.pallas_call(
        _fast_body,
        grid=(nb,),
        in_specs=[
            pl.BlockSpec((_Q, _D), lambda j: (0, 0)),
            pl.BlockSpec((_CB, _D), lambda j: (j, 0)),
        ],
        out_specs=[
            pl.BlockSpec((_Q, _K), lambda j: (0, 0)),
            pl.BlockSpec((_Q, _K), lambda j: (0, 0)),
            pl.BlockSpec((_Q, 128), lambda j: (0, 0)),
        ],
        out_shape=[
            jax.ShapeDtypeStruct((_Q, _K), jnp.float32),
            jax.ShapeDtypeStruct((_Q, _K), jnp.int32),
            jax.ShapeDtypeStruct((_Q, 128), jnp.int32),
        ],
        scratch_shapes=[
            pltpu.VMEM((_NCLS, _NRANK, _Q, 128), jnp.float32),
            pltpu.VMEM((_NCLS, 2, _Q, 128), jnp.int32),
        ],
        compiler_params=pltpu.CompilerParams(
            dimension_semantics=("arbitrary",),
            vmem_limit_bytes=63 * 1024 * 1024,
        ),
    )(qn, cn)


def _run_naive(qn, cn):
    nb = pl.cdiv(_N, _CB)
    vals, idx = pl.pallas_call(
        _naive_body,
        grid=(nb,),
        in_specs=[
            pl.BlockSpec((_Q, _D), lambda j: (0, 0)),
            pl.BlockSpec((_CB, _D), lambda j: (j, 0)),
        ],
        out_specs=[
            pl.BlockSpec((_Q, 128), lambda j: (0, 0)),
            pl.BlockSpec((_Q, 128), lambda j: (0, 0)),
        ],
        out_shape=[
            jax.ShapeDtypeStruct((_Q, 128), jnp.float32),
            jax.ShapeDtypeStruct((_Q, 128), jnp.int32),
        ],
        scratch_shapes=[
            pltpu.VMEM((_Q, 128), jnp.float32),
            pltpu.VMEM((_Q, 128), jnp.int32),
        ],
        compiler_params=pltpu.CompilerParams(
            dimension_semantics=("arbitrary",),
        ),
    )(qn, cn)
    return vals[:, :_K], idx[:, :_K]


@jax.jit
def kernel(queries, centroids, idf_weights):
    qw = queries * idf_weights[None, :]
    cw = centroids * idf_weights[None, :]
    qn = qw / (jnp.linalg.norm(qw, axis=-1, keepdims=True) + 1e-8)
    cn = cw / (jnp.linalg.norm(cw, axis=-1, keepdims=True) + 1e-8)

    fvals, fidx, flags = _run_fast(qn, cn)
    need_exact = jnp.max(flags) > 0
    return jax.lax.cond(
        need_exact,
        lambda qn, cn, fv, fi: _run_naive(qn, cn),
        lambda qn, cn, fv, fi: (fv, fi),
        qn, cn, fvals, fidx,
    )


# speedup vs baseline: 1.1484x; 1.1484x over previous
import jax
import jax.numpy as jnp
from jax.experimental import pallas as pl
from jax.experimental.pallas import tpu as pltpu

_K = 10
_Q = 1024
_D = 128
_N = 100000
_NEG = -3.0e38
_IHUGE = 2**31 - 1

_CB = 2048
_NCHUNK = _CB // 128
_NCLS = 4
_NRANK = 3
_QB = 128


def _fast_body(qn_ref, cn_ref, vals_out, idx_out, flag_out, tv, ti):
    j = pl.program_id(0)
    nb = pl.num_programs(0)

    @pl.when(j == 0)
    def _init():
        tv[...] = jnp.full(tv.shape, _NEG, jnp.float32)
        ti[...] = jnp.zeros(ti.shape, jnp.int32)

    qn = qn_ref[...]
    lane128 = jax.lax.broadcasted_iota(jnp.int32, (_Q, 128), 1)

    for p in range(_NCLS):
        panels = []
        for t in range(p, _NCHUNK, _NCLS):
            g = j * _NCHUNK + t
            cnc = cn_ref[pl.ds(t * 128, 128), :]
            v = jax.lax.dot_general(qn, cnc, (((1,), (1,)), ((), ())),
                                    preferred_element_type=jnp.float32)
            v = jnp.where(g * 128 + lane128 < _N, v, _NEG)
            panels.append((g, v))
        for qs in range(0, _Q, _QB):
            t1 = tv[p, 0, pl.ds(qs, _QB), :]
            t2 = tv[p, 1, pl.ds(qs, _QB), :]
            t3 = tv[p, 2, pl.ds(qs, _QB), :]
            i1 = ti[p, 0, pl.ds(qs, _QB), :]
            i2 = ti[p, 1, pl.ds(qs, _QB), :]
            for g, vfull in panels:
                v = vfull[qs:qs + _QB, :]
                gi = jnp.full((_QB, 128), g, jnp.int32)
                b1 = v > t1
                b2 = v > t2
                b3 = v > t3
                t3 = jnp.where(b2, t2, jnp.where(b3, v, t3))
                t2 = jnp.where(b1, t1, jnp.where(b2, v, t2))
                i2 = jnp.where(b1, i1, jnp.where(b2, gi, i2))
                t1 = jnp.where(b1, v, t1)
                i1 = jnp.where(b1, gi, i1)
            tv[p, 0, pl.ds(qs, _QB), :] = t1
            tv[p, 1, pl.ds(qs, _QB), :] = t2
            tv[p, 2, pl.ds(qs, _QB), :] = t3
            ti[p, 0, pl.ds(qs, _QB), :] = i1
            ti[p, 1, pl.ds(qs, _QB), :] = i2

    @pl.when(j == nb - 1)
    def _extract():
        lane = jax.lax.broadcasted_iota(jnp.int32, (_Q, 128), 1)
        cv = jnp.full((_Q, 128), _NEG, jnp.float32)
        ci = jnp.zeros((_Q, 128), jnp.int32)
        r = 0
        for p in range(_NCLS):
            for rank in range(2):
                av = tv[p, rank]
                ag = ti[p, rank] * 128 + lane
                for i in range(_K):
                    m = jnp.max(av, axis=1, keepdims=True)
                    eq = av == m
                    gidx = jnp.min(jnp.where(eq, ag, _IHUGE), axis=1,
                                   keepdims=True)
                    s = r * _K + i
                    cv = jnp.where(lane == s, m, cv)
                    ci = jnp.where(lane == s, gidx, ci)
                    av = jnp.where(eq & (ag == gidx), _NEG, av)
                r += 1
        ov = jnp.zeros((_Q, 128), jnp.float32)
        oi = jnp.zeros((_Q, 128), jnp.int32)
        m10 = None
        for i in range(_K):
            m = jnp.max(cv, axis=1, keepdims=True)
            eq = cv == m
            gidx = jnp.min(jnp.where(eq, ci, _IHUGE), axis=1, keepdims=True)
            ov = jnp.where(lane == i, m, ov)
            oi = jnp.where(lane == i, gidx, oi)
            cv = jnp.where(eq & (ci == gidx), _NEG, cv)
            m10 = m
        vals_out[...] = ov[:, :_K]
        idx_out[...] = oi[:, :_K]
        fl = jnp.zeros((_Q, 128), jnp.int32)
        for p in range(_NCLS):
            fl = fl | (tv[p, 2] >= m10).astype(jnp.int32)
        flag_out[...] = fl


def _naive_body(qn_ref, cn_ref, vals_out, idx_out, carry_v, carry_i):
    j = pl.program_id(0)
    nb = pl.num_programs(0)

    @pl.when(j == 0)
    def _init():
        carry_v[...] = jnp.full((_Q, 128), _NEG, jnp.float32)
        carry_i[...] = jnp.zeros((_Q, 128), jnp.int32)

    s = jax.lax.dot_general(qn_ref[...], cn_ref[...], (((1,), (1,)), ((), ())),
                            preferred_element_type=jnp.float32)
    col = j * _CB + jax.lax.broadcasted_iota(jnp.int32, (_Q, _CB), 1)
    s = jnp.where(col < _N, s, _NEG)

    lane128 = jax.lax.broadcasted_iota(jnp.int32, (_Q, 128), 1)
    bv = jnp.full((_Q, 128), _NEG, jnp.float32)
    bi = jnp.zeros((_Q, 128), jnp.int32)
    for i in range(_K):
        m = jnp.max(s, axis=1, keepdims=True)
        eq = s == m
        gidx = jnp.min(jnp.where(eq, col, _IHUGE), axis=1, keepdims=True)
        bv = jnp.where(lane128 == i, m, bv)
        bi = jnp.where(lane128 == i, gidx, bi)
        s = jnp.where(eq & (col == gidx), _NEG, s)

    mv = jnp.concatenate([carry_v[...], bv], axis=1)
    mi = jnp.concatenate([carry_i[...], bi], axis=1)
    nv = jnp.full((_Q, 128), _NEG, jnp.float32)
    ni = jnp.zeros((_Q, 128), jnp.int32)
    for i in range(_K):
        m = jnp.max(mv, axis=1, keepdims=True)
        eq = mv == m
        gidx = jnp.min(jnp.where(eq, mi, _IHUGE), axis=1, keepdims=True)
        nv = jnp.where(lane128 == i, m, nv)
        ni = jnp.where(lane128 == i, gidx, ni)
        mv = jnp.where(eq & (mi == gidx), _NEG, mv)
    carry_v[...] = nv
    carry_i[...] = ni

    @pl.when(j == nb - 1)
    def _finish():
        vals_out[...] = carry_v[...]
        idx_out[...] = carry_i[...]


def _run_fast(qn, cn):
    nb = pl.cdiv(_N, _CB)
    return pl.pallas_call(
        _fast_body,
        grid=(nb,),
        in_specs=[
            pl.BlockSpec((_Q, _D), lambda j: (0, 0)),
            pl.BlockSpec((_CB, _D), lambda j: (j, 0)),
        ],
        out_specs=[
            pl.BlockSpec((_Q, _K), lambda j: (0, 0)),
            pl.BlockSpec((_Q, _K), lambda j: (0, 0)),
            pl.BlockSpec((_Q, 128), lambda j: (0, 0)),
        ],
        out_shape=[
            jax.ShapeDtypeStruct((_Q, _K), jnp.float32),
            jax.ShapeDtypeStruct((_Q, _K), jnp.int32),
            jax.ShapeDtypeStruct((_Q, 128), jnp.int32),
        ],
        scratch_shapes=[
            pltpu.VMEM((_NCLS, _NRANK, _Q, 128), jnp.float32),
            pltpu.VMEM((_NCLS, 2, _Q, 128), jnp.int32),
        ],
        compiler_params=pltpu.CompilerParams(
            dimension_semantics=("arbitrary",),
            vmem_limit_bytes=63 * 1024 * 1024,
        ),
    )(qn, cn)


def _run_naive(qn, cn):
    nb = pl.cdiv(_N, _CB)
    vals, idx = pl.pallas_call(
        _naive_body,
        grid=(nb,),
        in_specs=[
            pl.BlockSpec((_Q, _D), lambda j: (0, 0)),
            pl.BlockSpec((_CB, _D), lambda j: (j, 0)),
        ],
        out_specs=[
            pl.BlockSpec((_Q, 128), lambda j: (0, 0)),
            pl.BlockSpec((_Q, 128), lambda j: (0, 0)),
        ],
        out_shape=[
            jax.ShapeDtypeStruct((_Q, 128), jnp.float32),
            jax.ShapeDtypeStruct((_Q, 128), jnp.int32),
        ],
        scratch_shapes=[
            pltpu.VMEM((_Q, 128), jnp.float32),
            pltpu.VMEM((_Q, 128), jnp.int32),
        ],
        compiler_params=pltpu.CompilerParams(
            dimension_semantics=("arbitrary",),
        ),
    )(qn, cn)
    return vals[:, :_K], idx[:, :_K]


@jax.jit
def kernel(queries, centroids, idf_weights):
    qw = queries * idf_weights[None, :]
    cw = centroids * idf_weights[None, :]
    qn = qw / (jnp.linalg.norm(qw, axis=-1, keepdims=True) + 1e-8)
    cn = cw / (jnp.linalg.norm(cw, axis=-1, keepdims=True) + 1e-8)

    fvals, fidx, flags = _run_fast(qn, cn)
    need_exact = jnp.max(flags) > 0
    return jax.lax.cond(
        need_exact,
        lambda qn, cn, fv, fi: _run_naive(qn, cn),
        lambda qn, cn, fv, fi: (fv, fi),
        qn, cn, fvals, fidx,
    )


# speedup vs baseline: 1.1625x; 1.0124x over previous
import jax
import jax.numpy as jnp
from jax.experimental import pallas as pl
from jax.experimental.pallas import tpu as pltpu

_K = 10
_Q = 1024
_D = 128
_N = 100000
_NEG = -3.0e38
_IHUGE = 2**31 - 1

_CB = 4096
_NCHUNK = _CB // 128
_NCLS = 4
_NRANK = 3
_QB = 128


def _fast_body(qn_ref, cn_ref, vals_out, idx_out, flag_out, tv, ti):
    j = pl.program_id(0)
    nb = pl.num_programs(0)

    @pl.when(j == 0)
    def _init():
        tv[...] = jnp.full(tv.shape, _NEG, jnp.float32)
        ti[...] = jnp.zeros(ti.shape, jnp.int32)

    qn = qn_ref[...]
    lane128 = jax.lax.broadcasted_iota(jnp.int32, (_Q, 128), 1)

    for p in range(_NCLS):
        panels = []
        for t in range(p, _NCHUNK, _NCLS):
            g = j * _NCHUNK + t
            cnc = cn_ref[pl.ds(t * 128, 128), :]
            v = jax.lax.dot_general(qn, cnc, (((1,), (1,)), ((), ())),
                                    preferred_element_type=jnp.float32)
            v = jnp.where(g * 128 + lane128 < _N, v, _NEG)
            panels.append((g, v))
        for qs in range(0, _Q, _QB):
            t1 = tv[p, 0, pl.ds(qs, _QB), :]
            t2 = tv[p, 1, pl.ds(qs, _QB), :]
            t3 = tv[p, 2, pl.ds(qs, _QB), :]
            i1 = ti[p, 0, pl.ds(qs, _QB), :]
            i2 = ti[p, 1, pl.ds(qs, _QB), :]
            for g, vfull in panels:
                v = vfull[qs:qs + _QB, :]
                gi = jnp.full((_QB, 128), g, jnp.int32)
                b1 = v > t1
                b2 = v > t2
                b3 = v > t3
                t3 = jnp.where(b2, t2, jnp.where(b3, v, t3))
                t2 = jnp.where(b1, t1, jnp.where(b2, v, t2))
                i2 = jnp.where(b1, i1, jnp.where(b2, gi, i2))
                t1 = jnp.where(b1, v, t1)
                i1 = jnp.where(b1, gi, i1)
            tv[p, 0, pl.ds(qs, _QB), :] = t1
            tv[p, 1, pl.ds(qs, _QB), :] = t2
            tv[p, 2, pl.ds(qs, _QB), :] = t3
            ti[p, 0, pl.ds(qs, _QB), :] = i1
            ti[p, 1, pl.ds(qs, _QB), :] = i2

    @pl.when(j == nb - 1)
    def _extract():
        lane = jax.lax.broadcasted_iota(jnp.int32, (_Q, 128), 1)
        cv = jnp.full((_Q, 128), _NEG, jnp.float32)
        ci = jnp.zeros((_Q, 128), jnp.int32)
        r = 0
        for p in range(_NCLS):
            for rank in range(2):
                av = tv[p, rank]
                ag = ti[p, rank] * 128 + lane
                for i in range(_K):
                    m = jnp.max(av, axis=1, keepdims=True)
                    eq = av == m
                    gidx = jnp.min(jnp.where(eq, ag, _IHUGE), axis=1,
                                   keepdims=True)
                    s = r * _K + i
                    cv = jnp.where(lane == s, m, cv)
                    ci = jnp.where(lane == s, gidx, ci)
                    av = jnp.where(eq & (ag == gidx), _NEG, av)
                r += 1
        ov = jnp.zeros((_Q, 128), jnp.float32)
        oi = jnp.zeros((_Q, 128), jnp.int32)
        m10 = None
        for i in range(_K):
            m = jnp.max(cv, axis=1, keepdims=True)
            eq = cv == m
            gidx = jnp.min(jnp.where(eq, ci, _IHUGE), axis=1, keepdims=True)
            ov = jnp.where(lane == i, m, ov)
            oi = jnp.where(lane == i, gidx, oi)
            cv = jnp.where(eq & (ci == gidx), _NEG, cv)
            m10 = m
        vals_out[...] = ov[:, :_K]
        idx_out[...] = oi[:, :_K]
        fl = jnp.zeros((_Q, 128), jnp.int32)
        for p in range(_NCLS):
            fl = fl | (tv[p, 2] >= m10).astype(jnp.int32)
        flag_out[...] = fl


def _naive_body(qn_ref, cn_ref, vals_out, idx_out, carry_v, carry_i):
    j = pl.program_id(0)
    nb = pl.num_programs(0)

    @pl.when(j == 0)
    def _init():
        carry_v[...] = jnp.full((_Q, 128), _NEG, jnp.float32)
        carry_i[...] = jnp.zeros((_Q, 128), jnp.int32)

    s = jax.lax.dot_general(qn_ref[...], cn_ref[...], (((1,), (1,)), ((), ())),
                            preferred_element_type=jnp.float32)
    col = j * _CB + jax.lax.broadcasted_iota(jnp.int32, (_Q, _CB), 1)
    s = jnp.where(col < _N, s, _NEG)

    lane128 = jax.lax.broadcasted_iota(jnp.int32, (_Q, 128), 1)
    bv = jnp.full((_Q, 128), _NEG, jnp.float32)
    bi = jnp.zeros((_Q, 128), jnp.int32)
    for i in range(_K):
        m = jnp.max(s, axis=1, keepdims=True)
        eq = s == m
        gidx = jnp.min(jnp.where(eq, col, _IHUGE), axis=1, keepdims=True)
        bv = jnp.where(lane128 == i, m, bv)
        bi = jnp.where(lane128 == i, gidx, bi)
        s = jnp.where(eq & (col == gidx), _NEG, s)

    mv = jnp.concatenate([carry_v[...], bv], axis=1)
    mi = jnp.concatenate([carry_i[...], bi], axis=1)
    nv = jnp.full((_Q, 128), _NEG, jnp.float32)
    ni = jnp.zeros((_Q, 128), jnp.int32)
    for i in range(_K):
        m = jnp.max(mv, axis=1, keepdims=True)
        eq = mv == m
        gidx = jnp.min(jnp.where(eq, mi, _IHUGE), axis=1, keepdims=True)
        nv = jnp.where(lane128 == i, m, nv)
        ni = jnp.where(lane128 == i, gidx, ni)
        mv = jnp.where(eq & (mi == gidx), _NEG, mv)
    carry_v[...] = nv
    carry_i[...] = ni

    @pl.when(j == nb - 1)
    def _finish():
        vals_out[...] = carry_v[...]
        idx_out[...] = carry_i[...]


def _run_fast(qn, cn):
    nb = pl.cdiv(_N, _CB)
    return pl.pallas_call(
        _fast_body,
        grid=(nb,),
        in_specs=[
            pl.BlockSpec((_Q, _D), lambda j: (0, 0)),
            pl.BlockSpec((_CB, _D), lambda j: (j, 0)),
        ],
        out_specs=[
            pl.BlockSpec((_Q, _K), lambda j: (0, 0)),
            pl.BlockSpec((_Q, _K), lambda j: (0, 0)),
            pl.BlockSpec((_Q, 128), lambda j: (0, 0)),
        ],
        out_shape=[
            jax.ShapeDtypeStruct((_Q, _K), jnp.float32),
            jax.ShapeDtypeStruct((_Q, _K), jnp.int32),
            jax.ShapeDtypeStruct((_Q, 128), jnp.int32),
        ],
        scratch_shapes=[
            pltpu.VMEM((_NCLS, _NRANK, _Q, 128), jnp.float32),
            pltpu.VMEM((_NCLS, 2, _Q, 128), jnp.int32),
        ],
        compiler_params=pltpu.CompilerParams(
            dimension_semantics=("arbitrary",),
            vmem_limit_bytes=63 * 1024 * 1024,
        ),
    )(qn, cn)


def _run_naive(qn, cn):
    nb = pl.cdiv(_N, _CB)
    vals, idx = pl.pallas_call(
        _naive_body,
        grid=(nb,),
        in_specs=[
            pl.BlockSpec((_Q, _D), lambda j: (0, 0)),
            pl.BlockSpec((_CB, _D), lambda j: (j, 0)),
        ],
        out_specs=[
            pl.BlockSpec((_Q, 128), lambda j: (0, 0)),
            pl.BlockSpec((_Q, 128), lambda j: (0, 0)),
        ],
        out_shape=[
            jax.ShapeDtypeStruct((_Q, 128), jnp.float32),
            jax.ShapeDtypeStruct((_Q, 128), jnp.int32),
        ],
        scratch_shapes=[
            pltpu.VMEM((_Q, 128), jnp.float32),
            pltpu.VMEM((_Q, 128), jnp.int32),
        ],
        compiler_params=pltpu.CompilerParams(
            dimension_semantics=("arbitrary",),
        ),
    )(qn, cn)
    return vals[:, :_K], idx[:, :_K]


@jax.jit
def kernel(queries, centroids, idf_weights):
    qw = queries * idf_weights[None, :]
    cw = centroids * idf_weights[None, :]
    qn = qw / (jnp.linalg.norm(qw, axis=-1, keepdims=True) + 1e-8)
    cn = cw / (jnp.linalg.norm(cw, axis=-1, keepdims=True) + 1e-8)

    fvals, fidx, flags = _run_fast(qn, cn)
    need_exact = jnp.max(flags) > 0
    return jax.lax.cond(
        need_exact,
        lambda qn, cn, fv, fi: _run_naive(qn, cn),
        lambda qn, cn, fv, fi: (fv, fi),
        qn, cn, fvals, fidx,
    )


# speedup vs baseline: 1.1880x; 1.0219x over previous
import jax
import jax.numpy as jnp
from jax.experimental import pallas as pl
from jax.experimental.pallas import tpu as pltpu

_K = 10
_Q = 1024
_D = 128
_N = 100000
_NEG = -3.0e38
_IHUGE = 2**31 - 1

_CB = 8192
_NCHUNK = _CB // 128
_NCLS = 4
_NRANK = 3
_QB = 128


def _fast_body(qn_ref, cn_ref, vals_out, idx_out, flag_out, tv, ti):
    j = pl.program_id(0)
    nb = pl.num_programs(0)

    @pl.when(j == 0)
    def _init():
        tv[...] = jnp.full(tv.shape, _NEG, jnp.float32)
        ti[...] = jnp.zeros(ti.shape, jnp.int32)

    qn = qn_ref[...]
    lane128 = jax.lax.broadcasted_iota(jnp.int32, (_Q, 128), 1)

    for p in range(_NCLS):
        panels = []
        for t in range(p, _NCHUNK, _NCLS):
            g = j * _NCHUNK + t
            cnc = cn_ref[pl.ds(t * 128, 128), :]
            v = jax.lax.dot_general(qn, cnc, (((1,), (1,)), ((), ())),
                                    preferred_element_type=jnp.float32)
            v = jnp.where(g * 128 + lane128 < _N, v, _NEG)
            panels.append((g, v))
        for qs in range(0, _Q, _QB):
            t1 = tv[p, 0, pl.ds(qs, _QB), :]
            t2 = tv[p, 1, pl.ds(qs, _QB), :]
            t3 = tv[p, 2, pl.ds(qs, _QB), :]
            i1 = ti[p, 0, pl.ds(qs, _QB), :]
            i2 = ti[p, 1, pl.ds(qs, _QB), :]
            for g, vfull in panels:
                v = vfull[qs:qs + _QB, :]
                gi = jnp.full((_QB, 128), g, jnp.int32)
                b1 = v > t1
                b2 = v > t2
                b3 = v > t3
                t3 = jnp.where(b2, t2, jnp.where(b3, v, t3))
                t2 = jnp.where(b1, t1, jnp.where(b2, v, t2))
                i2 = jnp.where(b1, i1, jnp.where(b2, gi, i2))
                t1 = jnp.where(b1, v, t1)
                i1 = jnp.where(b1, gi, i1)
            tv[p, 0, pl.ds(qs, _QB), :] = t1
            tv[p, 1, pl.ds(qs, _QB), :] = t2
            tv[p, 2, pl.ds(qs, _QB), :] = t3
            ti[p, 0, pl.ds(qs, _QB), :] = i1
            ti[p, 1, pl.ds(qs, _QB), :] = i2

    @pl.when(j == nb - 1)
    def _extract():
        lane = jax.lax.broadcasted_iota(jnp.int32, (_Q, 128), 1)
        cv = jnp.full((_Q, 128), _NEG, jnp.float32)
        ci = jnp.zeros((_Q, 128), jnp.int32)
        r = 0
        for p in range(_NCLS):
            for rank in range(2):
                av = tv[p, rank]
                ag = ti[p, rank] * 128 + lane
                for i in range(_K):
                    m = jnp.max(av, axis=1, keepdims=True)
                    eq = av == m
                    gidx = jnp.min(jnp.where(eq, ag, _IHUGE), axis=1,
                                   keepdims=True)
                    s = r * _K + i
                    cv = jnp.where(lane == s, m, cv)
                    ci = jnp.where(lane == s, gidx, ci)
                    av = jnp.where(eq & (ag == gidx), _NEG, av)
                r += 1
        ov = jnp.zeros((_Q, 128), jnp.float32)
        oi = jnp.zeros((_Q, 128), jnp.int32)
        m10 = None
        for i in range(_K):
            m = jnp.max(cv, axis=1, keepdims=True)
            eq = cv == m
            gidx = jnp.min(jnp.where(eq, ci, _IHUGE), axis=1, keepdims=True)
            ov = jnp.where(lane == i, m, ov)
            oi = jnp.where(lane == i, gidx, oi)
            cv = jnp.where(eq & (ci == gidx), _NEG, cv)
            m10 = m
        vals_out[...] = ov[:, :_K]
        idx_out[...] = oi[:, :_K]
        fl = jnp.zeros((_Q, 128), jnp.int32)
        for p in range(_NCLS):
            fl = fl | (tv[p, 2] >= m10).astype(jnp.int32)
        flag_out[...] = fl


def _naive_body(qn_ref, cn_ref, vals_out, idx_out, carry_v, carry_i):
    j = pl.program_id(0)
    nb = pl.num_programs(0)

    @pl.when(j == 0)
    def _init():
        carry_v[...] = jnp.full((_Q, 128), _NEG, jnp.float32)
        carry_i[...] = jnp.zeros((_Q, 128), jnp.int32)

    s = jax.lax.dot_general(qn_ref[...], cn_ref[...], (((1,), (1,)), ((), ())),
                            preferred_element_type=jnp.float32)
    col = j * _CB + jax.lax.broadcasted_iota(jnp.int32, (_Q, _CB), 1)
    s = jnp.where(col < _N, s, _NEG)

    lane128 = jax.lax.broadcasted_iota(jnp.int32, (_Q, 128), 1)
    bv = jnp.full((_Q, 128), _NEG, jnp.float32)
    bi = jnp.zeros((_Q, 128), jnp.int32)
    for i in range(_K):
        m = jnp.max(s, axis=1, keepdims=True)
        eq = s == m
        gidx = jnp.min(jnp.where(eq, col, _IHUGE), axis=1, keepdims=True)
        bv = jnp.where(lane128 == i, m, bv)
        bi = jnp.where(lane128 == i, gidx, bi)
        s = jnp.where(eq & (col == gidx), _NEG, s)

    mv = jnp.concatenate([carry_v[...], bv], axis=1)
    mi = jnp.concatenate([carry_i[...], bi], axis=1)
    nv = jnp.full((_Q, 128), _NEG, jnp.float32)
    ni = jnp.zeros((_Q, 128), jnp.int32)
    for i in range(_K):
        m = jnp.max(mv, axis=1, keepdims=True)
        eq = mv == m
        gidx = jnp.min(jnp.where(eq, mi, _IHUGE), axis=1, keepdims=True)
        nv = jnp.where(lane128 == i, m, nv)
        ni = jnp.where(lane128 == i, gidx, ni)
        mv = jnp.where(eq & (mi == gidx), _NEG, mv)
    carry_v[...] = nv
    carry_i[...] = ni

    @pl.when(j == nb - 1)
    def _finish():
        vals_out[...] = carry_v[...]
        idx_out[...] = carry_i[...]


def _run_fast(qn, cn):
    nb = pl.cdiv(_N, _CB)
    return pl.pallas_call(
        _fast_body,
        grid=(nb,),
        in_specs=[
            pl.BlockSpec((_Q, _D), lambda j: (0, 0)),
            pl.BlockSpec((_CB, _D), lambda j: (j, 0)),
        ],
        out_specs=[
            pl.BlockSpec((_Q, _K), lambda j: (0, 0)),
            pl.BlockSpec((_Q, _K), lambda j: (0, 0)),
            pl.BlockSpec((_Q, 128), lambda j: (0, 0)),
        ],
        out_shape=[
            jax.ShapeDtypeStruct((_Q, _K), jnp.float32),
            jax.ShapeDtypeStruct((_Q, _K), jnp.int32),
            jax.ShapeDtypeStruct((_Q, 128), jnp.int32),
        ],
        scratch_shapes=[
            pltpu.VMEM((_NCLS, _NRANK, _Q, 128), jnp.float32),
            pltpu.VMEM((_NCLS, 2, _Q, 128), jnp.int32),
        ],
        compiler_params=pltpu.CompilerParams(
            dimension_semantics=("arbitrary",),
            vmem_limit_bytes=63 * 1024 * 1024,
        ),
    )(qn, cn)


def _run_naive(qn, cn):
    nb = pl.cdiv(_N, _CB)
    vals, idx = pl.pallas_call(
        _naive_body,
        grid=(nb,),
        in_specs=[
            pl.BlockSpec((_Q, _D), lambda j: (0, 0)),
            pl.BlockSpec((_CB, _D), lambda j: (j, 0)),
        ],
        out_specs=[
            pl.BlockSpec((_Q, 128), lambda j: (0, 0)),
            pl.BlockSpec((_Q, 128), lambda j: (0, 0)),
        ],
        out_shape=[
            jax.ShapeDtypeStruct((_Q, 128), jnp.float32),
            jax.ShapeDtypeStruct((_Q, 128), jnp.int32),
        ],
        scratch_shapes=[
            pltpu.VMEM((_Q, 128), jnp.float32),
            pltpu.VMEM((_Q, 128), jnp.int32),
        ],
        compiler_params=pltpu.CompilerParams(
            dimension_semantics=("arbitrary",),
        ),
    )(qn, cn)
    return vals[:, :_K], idx[:, :_K]


@jax.jit
def kernel(queries, centroids, idf_weights):
    qw = queries * idf_weights[None, :]
    cw = centroids * idf_weights[None, :]
    qn = qw / (jnp.linalg.norm(qw, axis=-1, keepdims=True) + 1e-8)
    cn = cw / (jnp.linalg.norm(cw, axis=-1, keepdims=True) + 1e-8)

    fvals, fidx, flags = _run_fast(qn, cn)
    need_exact = jnp.max(flags) > 0
    return jax.lax.cond(
        need_exact,
        lambda qn, cn, fv, fi: _run_naive(qn, cn),
        lambda qn, cn, fv, fi: (fv, fi),
        qn, cn, fvals, fidx,
    )
